# baseline (speedup 1.0000x reference)
import jax
import jax.numpy as jnp
from jax import lax
from jax.experimental import pallas as pl
from jax.experimental.pallas import tpu as pltpu
from jax.experimental.pallas import tpu_sc as plsc

N_NODES = 10000
D = 128
E = 320000
N_HOPS_K = 3

NC = 2
NS = 16
NW = NC * NS
EPT = E // NW
C = 128
NCHUNK = 80
EPAD = NCHUNK * C
RPT = 624
ZB = 16
NZ = RPT // ZB
ZR = 104
TAIL = N_NODES - NS * RPT
TAIL_OFF = NS * RPT

RPC = 312
CTAIL_OFF = NW * RPC

_MESH = plsc.VectorSubcoreMesh(
    core_axis_name="c", subcore_axis_name="s", num_cores=NC, num_subcores=NS
)


def _hop_body(agg, pk, trf, part,
              ib0, ib1, ib2, ib3, tb0, tb1, tb2, tb3, gb0, gb1, zbuf, acc,
              is0, is1, is2, is3, ts0, ts1, ts2, ts3, gs0, gs1, ss0, ss1):
    cid = lax.axis_index("c")
    sid = lax.axis_index("s")
    wid = cid * NS + sid

    ib = (ib0, ib1, ib2, ib3)
    isem = (is0, is1, is2, is3)
    tb = (tb0, tb1, tb2, tb3)
    tsem = (ts0, ts1, ts2, ts3)
    gb = (gb0, gb1)
    gsem = (gs0, gs1)
    ssem = (ss0, ss1)

    def idx_load(c, s):
        pltpu.async_copy(pk.at[wid, c], ib[s], isem[s])
        pltpu.async_copy(trf.at[wid, c], tb[s], tsem[s])

    def idx_wait(s):
        pltpu.make_async_copy(pk.at[wid, 0], ib[s], isem[s]).wait()
        pltpu.make_async_copy(trf.at[wid, 0], tb[s], tsem[s]).wait()

    def gather_start(s, p):
        pltpu.async_copy(agg.at[ib[s].at[0]], gb[p], gsem[p])

    def gather_wait(s, p):
        pltpu.make_async_copy(agg.at[ib[s].at[0]], gb[p], gsem[p]).wait()

    def scatter_start(s, p):
        pltpu.async_copy(gb[p], acc.at[ib[s].at[1]], ssem[p], add=True)

    def scatter_wait(s, p):
        pltpu.make_async_copy(gb[p], acc.at[ib[s].at[1]], ssem[p]).wait()

    def scale(s, p):
        buf = gb[p]
        tr_ref = tb[s]

        def grp(j16, carry):
            t16 = tr_ref[0, pl.ds(j16 * 16, 16)]
            for jj in range(16):
                tb = lax.broadcast(t16[jj], (16,))
                j = j16 * 16 + jj
                for k in range(D // 16):
                    buf[j, pl.ds(k * 16, 16)] = buf[j, pl.ds(k * 16, 16)] * tb
            return carry
        lax.fori_loop(0, C // 16, grp, 0)

    def zb(j, carry):
        for k in range(D // 16):
            zbuf[j, pl.ds(k * 16, 16)] = jnp.zeros((16,), jnp.float32)
        return carry
    lax.fori_loop(0, ZB, zb, 0)

    def za(k, carry):
        pltpu.sync_copy(zbuf, acc.at[pl.ds(sid * RPT + k * ZB, ZB)])
        return carry
    lax.fori_loop(0, NZ, za, 0)

    @pl.when(sid == NS - 1)
    def _():
        pltpu.sync_copy(zbuf.at[pl.ds(0, TAIL)], acc.at[pl.ds(TAIL_OFF, TAIL)])
    plsc.subcore_barrier()

    idx_load(0, 0)
    idx_load(1, 1)
    idx_load(2, 2)
    idx_wait(0)
    gather_start(0, 0)

    def steady(c, k, guard):
        sp, pp = (k + 3) % 4, (k + 1) % 2
        scatter_wait(sp, pp)
        if guard:
            @pl.when(c + 3 < NCHUNK)
            def _():
                idx_load(c + 3, (k + 3) % 4)

            @pl.when(c + 1 < NCHUNK)
            def _():
                idx_wait((k + 1) % 4)
                gather_start((k + 1) % 4, (k + 1) % 2)
        else:
            idx_load(c + 3, (k + 3) % 4)
            idx_wait((k + 1) % 4)
            gather_start((k + 1) % 4, (k + 1) % 2)
        gather_wait(k, k % 2)
        scale(k, k % 2)
        scatter_start(k, k % 2)

    idx_load(3, 3)
    idx_wait(1)
    gather_start(1, 1)
    gather_wait(0, 0)
    scale(0, 0)
    scatter_start(0, 0)
    steady(1, 1, False)
    steady(2, 2, False)
    steady(3, 3, False)

    def quad(i4, carry):
        c = 4 * i4
        steady(c + 0, 0, True)
        steady(c + 1, 1, True)
        steady(c + 2, 2, True)
        steady(c + 3, 3, True)
        return carry
    lax.fori_loop(1, NCHUNK // 4, quad, 0)

    scatter_wait(3, 1)

    plsc.subcore_barrier()
    pltpu.sync_copy(acc.at[pl.ds(sid * RPT, RPT)],
                    part.at[cid, pl.ds(sid * RPT, RPT)])

    @pl.when(sid == NS - 1)
    def _():
        pltpu.sync_copy(acc.at[pl.ds(TAIL_OFF, TAIL)],
                        part.at[cid, pl.ds(TAIL_OFF, TAIL)])


def _combine_body(part, out, b0, b1):
    cid = lax.axis_index("c")
    sid = lax.axis_index("s")
    wid = cid * NS + sid

    def _sum_rows(nrows, off):
        pltpu.sync_copy(part.at[0, pl.ds(off, nrows)], b0.at[pl.ds(0, nrows)])
        pltpu.sync_copy(part.at[1, pl.ds(off, nrows)], b1.at[pl.ds(0, nrows)])

        def addrow(j, c2):
            for kk in range(D // 16):
                b0[j, pl.ds(kk * 16, 16)] = (
                    b0[j, pl.ds(kk * 16, 16)] + b1[j, pl.ds(kk * 16, 16)])
            return c2
        lax.fori_loop(0, nrows, addrow, 0)
        pltpu.sync_copy(b0.at[pl.ds(0, nrows)], out.at[pl.ds(off, nrows)])

    def ck(k, carry):
        _sum_rows(ZR, wid * RPC + k * ZR)
        return carry
    lax.fori_loop(0, RPC // ZR, ck, 0)

    @pl.when(wid == NW - 1)
    def _():
        _sum_rows(TAIL, CTAIL_OFF)


_hop = pl.kernel(
    _hop_body,
    out_type=jax.ShapeDtypeStruct((NC, N_NODES, D), jnp.float32),
    mesh=_MESH,
    scratch_types=[
        pltpu.VMEM((2, C), jnp.int32),
        pltpu.VMEM((2, C), jnp.int32),
        pltpu.VMEM((2, C), jnp.int32),
        pltpu.VMEM((2, C), jnp.int32),
        pltpu.VMEM((1, C), jnp.float32),
        pltpu.VMEM((1, C), jnp.float32),
        pltpu.VMEM((1, C), jnp.float32),
        pltpu.VMEM((1, C), jnp.float32),
        pltpu.VMEM((C, D), jnp.float32),
        pltpu.VMEM((C, D), jnp.float32),
        pltpu.VMEM((ZB, D), jnp.float32),
        pltpu.VMEM_SHARED((N_NODES, D), jnp.float32),
        pltpu.SemaphoreType.DMA,
        pltpu.SemaphoreType.DMA,
        pltpu.SemaphoreType.DMA,
        pltpu.SemaphoreType.DMA,
        pltpu.SemaphoreType.DMA,
        pltpu.SemaphoreType.DMA,
        pltpu.SemaphoreType.DMA,
        pltpu.SemaphoreType.DMA,
        pltpu.SemaphoreType.DMA,
        pltpu.SemaphoreType.DMA,
        pltpu.SemaphoreType.DMA,
        pltpu.SemaphoreType.DMA,
    ],
)

_combine = pl.kernel(
    _combine_body,
    out_type=jax.ShapeDtypeStruct((N_NODES, D), jnp.float32),
    mesh=_MESH,
    scratch_types=[
        pltpu.VMEM((ZR, D), jnp.float32),
        pltpu.VMEM((ZR, D), jnp.float32),
    ],
)


def _pad_chunks(x):
    x = x.reshape(NW, EPT)
    x = jnp.pad(x, ((0, 0), (0, EPAD - EPT)))
    return x.reshape(NW, NCHUNK, 1, C)


def kernel(embed, edge_index, trend):
    row = _pad_chunks(edge_index[0].astype(jnp.int32))
    col = _pad_chunks(edge_index[1].astype(jnp.int32))
    trf = _pad_chunks(trend.astype(jnp.float32))
    pk = jnp.concatenate([row, col], axis=2)

    embs = [embed]
    agg = embed
    for _ in range(N_HOPS_K):
        part = _hop(agg, pk, trf)
        agg = _combine(part)
        embs.append(agg)
    return jnp.stack(embs, axis=1)


# speedup vs baseline: 1.0034x; 1.0034x over previous
import jax
import jax.numpy as jnp
from jax import lax
from jax.experimental import pallas as pl
from jax.experimental.pallas import tpu as pltpu
from jax.experimental.pallas import tpu_sc as plsc

N_NODES = 10000
D = 128
E = 320000
N_HOPS_K = 3

NC = 2
NS = 16
NW = NC * NS
EPT = E // NW
C = 128
NCHUNK = 80
EPAD = NCHUNK * C
BLK = 16
NBLK = NCHUNK // BLK
RPT = 624
ZB = 16
NZ = RPT // ZB
ZR = 104
TAIL = N_NODES - NS * RPT
TAIL_OFF = NS * RPT

RPC = 312
CTAIL_OFF = NW * RPC

_MESH = plsc.VectorSubcoreMesh(
    core_axis_name="c", subcore_axis_name="s", num_cores=NC, num_subcores=NS
)


def _hop_body(agg, pk, trf, part,
              pkq0, pkq1, trq0, trq1, rb0, rb1, cb0, cb1, tf0, tf1,
              gb0, gb1, zbuf, acc,
              qs0, qs1, gs0, gs1, ss0, ss1):
    cid = lax.axis_index("c")
    sid = lax.axis_index("s")
    wid = cid * NS + sid

    pkq = (pkq0, pkq1)
    trq = (trq0, trq1)
    rb = (rb0, rb1)
    cb = (cb0, cb1)
    tf = (tf0, tf1)
    gb = (gb0, gb1)
    qsem = (qs0, qs1)
    gsem = (gs0, gs1)
    ssem = (ss0, ss1)

    def zb(j, carry):
        for k in range(D // 16):
            zbuf[j, pl.ds(k * 16, 16)] = jnp.zeros((16,), jnp.float32)
        return carry
    lax.fori_loop(0, ZB, zb, 0)

    def za(k, carry):
        pltpu.sync_copy(zbuf, acc.at[pl.ds(sid * RPT + k * ZB, ZB)])
        return carry
    lax.fori_loop(0, NZ, za, 0)

    @pl.when(sid == NS - 1)
    def _():
        pltpu.sync_copy(zbuf.at[pl.ds(0, TAIL)], acc.at[pl.ds(TAIL_OFF, TAIL)])
    plsc.subcore_barrier()

    pltpu.sync_copy(pk.at[wid, 0], pkq0)
    pltpu.sync_copy(trf.at[wid, 0], trq0)
    pltpu.async_copy(pk.at[wid, 1], pkq1, qs1)
    pltpu.async_copy(trf.at[wid, 1], trq1, qs1)

    def unpack(cc, p):
        blk = cc // BLK
        lc = cc - blk * BLK
        qsel = lax.rem(blk, 2)
        for q in range(2):
            @pl.when(qsel == q)
            def _():
                for w in range(C // 16):
                    v = pkq[q][lc, pl.ds(w * 16, 16)]
                    rb[p][pl.ds(w * 16, 16)] = v & 0xFFFF
                    cb[p][pl.ds(w * 16, 16)] = lax.shift_right_logical(v, 16)
                    tf[p][pl.ds(w * 16, 16)] = trq[q][lc, pl.ds(w * 16, 16)]

    def gather_start(p):
        pltpu.async_copy(agg.at[rb[p]], gb[p], gsem[p])

    def gather_wait(p):
        pltpu.make_async_copy(agg.at[rb[p]], gb[p], gsem[p]).wait()

    def scatter_start(p):
        pltpu.async_copy(gb[p], acc.at[cb[p]], ssem[p], add=True)

    def scatter_wait(p):
        pltpu.make_async_copy(gb[p], acc.at[cb[p]], ssem[p]).wait()

    def scale(p):
        buf = gb[p]
        tr_ref = tf[p]

        def grp(j16, carry):
            t16 = tr_ref[pl.ds(j16 * 16, 16)]
            for jj in range(16):
                tbc = lax.broadcast(t16[jj], (16,))
                j = j16 * 16 + jj
                for k in range(D // 16):
                    buf[j, pl.ds(k * 16, 16)] = buf[j, pl.ds(k * 16, 16)] * tbc
            return carry
        lax.fori_loop(0, C // 16, grp, 0)

    def blkmgmt(c):
        blk = c // BLK
        nq = lax.rem(blk + 1, 2)

        @pl.when(jnp.logical_and(c - blk * BLK == 2,
                                 jnp.logical_and(c > BLK, c < 66)))
        def _():
            for q in range(2):
                @pl.when(nq == q)
                def _():
                    pltpu.async_copy(pk.at[wid, blk + 1], pkq[q], qsem[q])
                    pltpu.async_copy(trf.at[wid, blk + 1], trq[q], qsem[q])

        @pl.when(jnp.logical_and(c - blk * BLK == 14, c < 64))
        def _():
            for q in range(2):
                @pl.when(nq == q)
                def _():
                    pltpu.make_async_copy(pk.at[wid, 0], pkq[q], qsem[q]).wait()
                    pltpu.make_async_copy(trf.at[wid, 0], trq[q], qsem[q]).wait()

    def pairbody(i2, carry):
        c = 2 * i2

        @pl.when(i2 == 0)
        def _():
            unpack(0, 0)
            gather_start(0)

        @pl.when(i2 > 0)
        def _():
            scatter_wait(1)
        blkmgmt(c)
        unpack(c + 1, 1)
        gather_start(1)
        gather_wait(0)
        scale(0)
        scatter_start(0)

        scatter_wait(0)
        blkmgmt(c + 1)

        @pl.when(c + 2 < NCHUNK)
        def _():
            unpack(c + 2, 0)
            gather_start(0)
        gather_wait(1)
        scale(1)
        scatter_start(1)
        return carry
    lax.fori_loop(0, NCHUNK // 2, pairbody, 0)
    scatter_wait(1)

    plsc.subcore_barrier()
    pltpu.sync_copy(acc.at[pl.ds(sid * RPT, RPT)],
                    part.at[cid, pl.ds(sid * RPT, RPT)])

    @pl.when(sid == NS - 1)
    def _():
        pltpu.sync_copy(acc.at[pl.ds(TAIL_OFF, TAIL)],
                        part.at[cid, pl.ds(TAIL_OFF, TAIL)])


def _combine_body(part, out, b0, b1):
    cid = lax.axis_index("c")
    sid = lax.axis_index("s")
    wid = cid * NS + sid

    def _sum_rows(nrows, off):
        pltpu.sync_copy(part.at[0, pl.ds(off, nrows)], b0.at[pl.ds(0, nrows)])
        pltpu.sync_copy(part.at[1, pl.ds(off, nrows)], b1.at[pl.ds(0, nrows)])

        def addrow(j, c2):
            for kk in range(D // 16):
                b0[j, pl.ds(kk * 16, 16)] = (
                    b0[j, pl.ds(kk * 16, 16)] + b1[j, pl.ds(kk * 16, 16)])
            return c2
        lax.fori_loop(0, nrows, addrow, 0)
        pltpu.sync_copy(b0.at[pl.ds(0, nrows)], out.at[pl.ds(off, nrows)])

    def ck(k, carry):
        _sum_rows(ZR, wid * RPC + k * ZR)
        return carry
    lax.fori_loop(0, RPC // ZR, ck, 0)

    @pl.when(wid == NW - 1)
    def _():
        _sum_rows(TAIL, CTAIL_OFF)


_hop = pl.kernel(
    _hop_body,
    out_type=jax.ShapeDtypeStruct((NC, N_NODES, D), jnp.float32),
    mesh=_MESH,
    scratch_types=[
        pltpu.VMEM((BLK, C), jnp.int32),
        pltpu.VMEM((BLK, C), jnp.int32),
        pltpu.VMEM((BLK, C), jnp.float32),
        pltpu.VMEM((BLK, C), jnp.float32),
        pltpu.VMEM((C,), jnp.int32),
        pltpu.VMEM((C,), jnp.int32),
        pltpu.VMEM((C,), jnp.int32),
        pltpu.VMEM((C,), jnp.int32),
        pltpu.VMEM((C,), jnp.float32),
        pltpu.VMEM((C,), jnp.float32),
        pltpu.VMEM((C, D), jnp.float32),
        pltpu.VMEM((C, D), jnp.float32),
        pltpu.VMEM((ZB, D), jnp.float32),
        pltpu.VMEM_SHARED((N_NODES, D), jnp.float32),
        pltpu.SemaphoreType.DMA,
        pltpu.SemaphoreType.DMA,
        pltpu.SemaphoreType.DMA,
        pltpu.SemaphoreType.DMA,
        pltpu.SemaphoreType.DMA,
        pltpu.SemaphoreType.DMA,
    ],
)

_combine = pl.kernel(
    _combine_body,
    out_type=jax.ShapeDtypeStruct((N_NODES, D), jnp.float32),
    mesh=_MESH,
    scratch_types=[
        pltpu.VMEM((ZR, D), jnp.float32),
        pltpu.VMEM((ZR, D), jnp.float32),
    ],
)


def _pad_chunks(x):
    x = x.reshape(NW, EPT)
    x = jnp.pad(x, ((0, 0), (0, EPAD - EPT)))
    return x.reshape(NW, NBLK, BLK, C)


def kernel(embed, edge_index, trend):
    row = edge_index[0].astype(jnp.int32)
    col = edge_index[1].astype(jnp.int32)
    pk = _pad_chunks(row | (col << 16))
    trf = _pad_chunks(trend.astype(jnp.float32))

    embs = [embed]
    agg = embed
    for _ in range(N_HOPS_K):
        part = _hop(agg, pk, trf)
        agg = _combine(part)
        embs.append(agg)
    return jnp.stack(embs, axis=1)
